# Initial kernel scaffold; baseline (speedup 1.0000x reference)
import functools

import jax
import jax.numpy as jnp
from jax import lax
from jax.experimental import pallas as pl
from jax.experimental.pallas import tpu as pltpu
from jax.experimental.pallas import tpu_sc as plsc

N_NODES = 10000
N_EDGES = 160000
D_IN = 256
D_HID = 512

TILES = 16
BATCH = 128
NB = 79
E_PAD = TILES * NB * BATCH
N_PAD = 10240
ROWS_PER_TILE = N_PAD // TILES
CHUNK = 128

_mesh = plsc.VectorSubcoreMesh(core_axis_name="c", subcore_axis_name="s")


def _fill_zero(ref, rows, cols):
    def body(i, _):
        for k in range(cols // 16):
            ref[i, pl.ds(k * 16, 16)] = jnp.zeros((16,), jnp.float32)
        return 0
    lax.fori_loop(0, rows, body, 0)


def _fill_one(ref, rows, cols):
    def body(i, _):
        for k in range(cols // 16):
            ref[i, pl.ds(k * 16, 16)] = jnp.ones((16,), jnp.float32)
        return 0
    lax.fori_loop(0, rows, body, 0)


def _agg_pass(tbl_hbm, out_hbm, src_v, dst_v, rows_v, zbuf, acc_sp,
              onesb, cnt_sp, cnt_hbm, sub):
    base = sub * ROWS_PER_TILE
    for k in range(ROWS_PER_TILE // BATCH):
        pltpu.sync_copy(zbuf, acc_sp.at[pl.ds(base + k * BATCH, BATCH)])
    if cnt_sp is not None:
        pltpu.sync_copy(zbuf.at[:, pl.ds(0, 16)],
                        cnt_sp.at[pl.ds(base, BATCH)])
        for k in range(1, ROWS_PER_TILE // BATCH):
            pltpu.sync_copy(zbuf.at[:, pl.ds(0, 16)],
                            cnt_sp.at[pl.ds(base + k * BATCH, BATCH)])
    plsc.subcore_barrier()

    def body(j, _):
        pltpu.sync_copy(tbl_hbm.at[src_v.at[j]], rows_v)
        pltpu.sync_copy(rows_v, acc_sp.at[dst_v.at[j]], add=True)
        if cnt_sp is not None:
            pltpu.sync_copy(onesb, cnt_sp.at[dst_v.at[j]], add=True)
        return 0
    lax.fori_loop(0, NB, body, 0)

    plsc.subcore_barrier()
    pltpu.sync_copy(acc_sp.at[pl.ds(base, ROWS_PER_TILE)],
                    out_hbm.at[pl.ds(base, ROWS_PER_TILE)])
    if cnt_sp is not None:
        pltpu.sync_copy(cnt_sp.at[pl.ds(base, ROWS_PER_TILE)],
                        cnt_hbm.at[pl.ds(base, ROWS_PER_TILE)])


def _seg_body(chunks_per_core, with_cnt, *refs):
    n_tbl = 2 * chunks_per_core
    tbls = refs[:n_tbl]
    srcT, dstT = refs[n_tbl], refs[n_tbl + 1]
    outs = refs[n_tbl + 2:2 * n_tbl + 2]
    rest = refs[2 * n_tbl + 2:]
    if with_cnt:
        cnt_hbm = rest[0]
        src_v, dst_v, rows_v, zbuf, onesb, acc_sp, cnt_sp = rest[1:]
    else:
        cnt_hbm = None
        src_v, dst_v, rows_v, zbuf, acc_sp = rest
        onesb = cnt_sp = None

    core = lax.axis_index("c")
    sub = lax.axis_index("s")

    _fill_zero(zbuf, BATCH, CHUNK)
    if with_cnt:
        _fill_one(onesb, BATCH, 16)
    pltpu.sync_copy(srcT.at[sub], src_v)
    pltpu.sync_copy(dstT.at[sub], dst_v)

    @pl.when(core == 0)
    def _():
        for ci in range(chunks_per_core):
            _agg_pass(tbls[ci], outs[ci], src_v, dst_v, rows_v, zbuf,
                      acc_sp, onesb, cnt_sp if ci == 0 else None,
                      cnt_hbm, sub)

    @pl.when(core == 1)
    def _():
        for ci in range(chunks_per_core):
            _agg_pass(tbls[chunks_per_core + ci], outs[chunks_per_core + ci],
                      src_v, dst_v, rows_v, zbuf, acc_sp,
                      None, None, None, sub)


def _make_seg_kernel(chunks_per_core, with_cnt):
    n_tbl = 2 * chunks_per_core
    out_type = [jax.ShapeDtypeStruct((N_PAD, CHUNK), jnp.float32)
                for _ in range(n_tbl)]
    if with_cnt:
        out_type.append(jax.ShapeDtypeStruct((N_PAD, 16), jnp.float32))
    scratch = [
        pltpu.VMEM((NB, BATCH), jnp.int32),
        pltpu.VMEM((NB, BATCH), jnp.int32),
        pltpu.VMEM((BATCH, CHUNK), jnp.float32),
        pltpu.VMEM((BATCH, CHUNK), jnp.float32),
    ]
    if with_cnt:
        scratch.append(pltpu.VMEM((BATCH, 16), jnp.float32))
    scratch.append(pltpu.VMEM_SHARED((N_PAD, CHUNK), jnp.float32))
    if with_cnt:
        scratch.append(pltpu.VMEM_SHARED((N_PAD, 16), jnp.float32))
    return pl.kernel(
        functools.partial(_seg_body, chunks_per_core, with_cnt),
        out_type=tuple(out_type),
        mesh=_mesh,
        scratch_types=tuple(scratch),
    )


_seg_l1 = _make_seg_kernel(1, True)
_seg_l2 = _make_seg_kernel(2, False)


M_BLK = 400


def _mm_body(relu, s_ref, cnt_ref, x_ref, wl_ref, wr_ref, b_ref, o_ref):
    inv = 1.0 / jnp.maximum(cnt_ref[...], 1.0)
    mean = s_ref[...] * inv
    acc = jnp.dot(mean, wl_ref[...], preferred_element_type=jnp.float32)
    acc = acc + jnp.dot(x_ref[...], wr_ref[...],
                        preferred_element_type=jnp.float32)
    acc = acc + b_ref[...]
    if relu:
        acc = jnp.maximum(acc, 0.0)
    o_ref[...] = acc


def _make_mm(d_in, relu):
    grid = N_NODES // M_BLK
    return pl.pallas_call(
        functools.partial(_mm_body, relu),
        grid=(grid,),
        in_specs=[
            pl.BlockSpec((M_BLK, d_in), lambda i: (i, 0)),
            pl.BlockSpec((M_BLK, 1), lambda i: (i, 0)),
            pl.BlockSpec((M_BLK, d_in), lambda i: (i, 0)),
            pl.BlockSpec((d_in, D_HID), lambda i: (0, 0)),
            pl.BlockSpec((d_in, D_HID), lambda i: (0, 0)),
            pl.BlockSpec((1, D_HID), lambda i: (0, 0)),
        ],
        out_specs=pl.BlockSpec((M_BLK, D_HID), lambda i: (i, 0)),
        out_shape=jax.ShapeDtypeStruct((N_NODES, D_HID), jnp.float32),
    )


_mm1 = _make_mm(D_IN, True)
_mm2 = _make_mm(D_HID, False)


def kernel(x, edge_index, Wl1, bl1, Wr1, Wl2, bl2, Wr2):
    src = edge_index[0].astype(jnp.int32)
    dst = edge_index[1].astype(jnp.int32)
    pad = E_PAD - N_EDGES
    srcT = jnp.concatenate([src, jnp.zeros((pad,), jnp.int32)]
                           ).reshape(TILES, NB, BATCH)
    dstT = jnp.concatenate([dst, jnp.full((pad,), N_NODES, jnp.int32)]
                           ).reshape(TILES, NB, BATCH)

    xa = x[:, :CHUNK]
    xb = x[:, CHUNK:]
    sa, sb, cntw = _seg_l1(xa, xb, srcT, dstT)
    s1 = jnp.concatenate([sa[:N_NODES], sb[:N_NODES]], axis=1)
    cnt = cntw[:N_NODES, :1]

    h = _mm1(s1, cnt, x, Wl1.T, Wr1.T, bl1.reshape(1, -1))

    hs = [h[:, i * CHUNK:(i + 1) * CHUNK] for i in range(4)]
    t0, t1, t2, t3 = _seg_l2(hs[0], hs[1], hs[2], hs[3], srcT, dstT)
    s2 = jnp.concatenate([t0[:N_NODES], t1[:N_NODES],
                          t2[:N_NODES], t3[:N_NODES]], axis=1)

    return _mm2(s2, cnt, h, Wl2.T, Wr2.T, bl2.reshape(1, -1))


# speedup vs baseline: 2.7720x; 2.7720x over previous
import functools

import jax
import jax.numpy as jnp
from jax import lax
from jax.experimental import pallas as pl
from jax.experimental.pallas import tpu as pltpu
from jax.experimental.pallas import tpu_sc as plsc

N_NODES = 10000
N_EDGES = 160000
D_IN = 256
D_HID = 512

TILES = 16
BATCH = 128
NB = 80
W = 8
E_PAD = TILES * NB * BATCH
N_PAD = 10240
ROWS_PER_TILE = N_PAD // TILES
CHUNK = 128
ZROWS = 32


def _fill_zero(ref, rows, cols):
    def body(i, _):
        for k in range(cols // 16):
            ref[i, pl.ds(k * 16, 16)] = jnp.zeros((16,), jnp.float32)
        return 0
    lax.fori_loop(0, rows, body, 0)


def _agg_pass(tbl_hbm, out_hbm, srcT, dstT, src_w, dst_w, rows_v, zbuf,
              acc_sp, sub):
    base = sub * ROWS_PER_TILE
    for k in range(ROWS_PER_TILE // ZROWS):
        pltpu.sync_copy(zbuf, acc_sp.at[pl.ds(base + k * ZROWS, ZROWS)])
    plsc.subcore_barrier()

    def win(w, _):
        pltpu.sync_copy(srcT.at[sub, pl.ds(w * W, W)], src_w)
        pltpu.sync_copy(dstT.at[sub, pl.ds(w * W, W)], dst_w)
        for b in range(W):
            pltpu.sync_copy(tbl_hbm.at[src_w.at[b]], rows_v)
            pltpu.sync_copy(rows_v, acc_sp.at[dst_w.at[b]], add=True)
        return 0
    lax.fori_loop(0, NB // W, win, 0)

    plsc.subcore_barrier()
    pltpu.sync_copy(acc_sp.at[pl.ds(base, ROWS_PER_TILE)],
                    out_hbm.at[pl.ds(base, ROWS_PER_TILE)])


def _count_pass(dstT, dst_w, cnt_local, cnt_hbm, sub):
    def zb(i, _):
        cnt_local[pl.ds(i * 16, 16)] = jnp.zeros((16,), jnp.float32)
        return 0
    lax.fori_loop(0, N_PAD // 16, zb, 0)

    ones16 = jnp.ones((16,), jnp.float32)

    def win(w, _):
        pltpu.sync_copy(dstT.at[sub, pl.ds(w * W, W)], dst_w)
        for b in range(W):
            for k in range(BATCH // 16):
                d = dst_w[b, pl.ds(k * 16, 16)]
                plsc.addupdate_scatter(cnt_local, [d], ones16)
        return 0
    lax.fori_loop(0, NB // W, win, 0)

    pltpu.sync_copy(cnt_local, cnt_hbm.at[sub])


def _seg_body(chunks_per_core, with_cnt, *refs):
    n_tbl = 2 * chunks_per_core
    tbls = refs[:n_tbl]
    srcT, dstT = refs[n_tbl], refs[n_tbl + 1]
    outs = refs[n_tbl + 2:2 * n_tbl + 2]
    rest = refs[2 * n_tbl + 2:]
    if with_cnt:
        cnt_hbm = rest[0]
        src_w, dst_w, rows_v, zbuf, cnt_local, acc_sp = rest[1:]
    else:
        cnt_hbm = None
        src_w, dst_w, rows_v, zbuf, acc_sp = rest
        cnt_local = None

    core = lax.axis_index("c")
    sub = lax.axis_index("s")

    _fill_zero(zbuf, ZROWS, CHUNK)

    @pl.when(core == 0)
    def _():
        for ci in range(chunks_per_core):
            _agg_pass(tbls[ci], outs[ci], srcT, dstT, src_w, dst_w,
                      rows_v, zbuf, acc_sp, sub)
        if with_cnt:
            _count_pass(dstT, dst_w, cnt_local, cnt_hbm, sub)

    @pl.when(core == 1)
    def _():
        for ci in range(chunks_per_core):
            _agg_pass(tbls[chunks_per_core + ci], outs[chunks_per_core + ci],
                      srcT, dstT, src_w, dst_w, rows_v, zbuf, acc_sp, sub)


@functools.lru_cache(maxsize=None)
def _make_seg_kernel(chunks_per_core, with_cnt):
    mesh = plsc.VectorSubcoreMesh(core_axis_name="c", subcore_axis_name="s")
    n_tbl = 2 * chunks_per_core
    out_type = [jax.ShapeDtypeStruct((N_PAD, CHUNK), jnp.float32)
                for _ in range(n_tbl)]
    if with_cnt:
        out_type.append(jax.ShapeDtypeStruct((TILES, N_PAD), jnp.float32))
    scratch = [
        pltpu.VMEM((W, BATCH), jnp.int32),
        pltpu.VMEM((W, BATCH), jnp.int32),
        pltpu.VMEM((BATCH, CHUNK), jnp.float32),
        pltpu.VMEM((ZROWS, CHUNK), jnp.float32),
    ]
    if with_cnt:
        scratch.append(pltpu.VMEM((N_PAD,), jnp.float32))
    scratch.append(pltpu.VMEM_SHARED((N_PAD, CHUNK), jnp.float32))
    return pl.kernel(
        functools.partial(_seg_body, chunks_per_core, with_cnt),
        out_type=tuple(out_type),
        mesh=mesh,
        scratch_types=tuple(scratch),
        compiler_params=pltpu.CompilerParams(needs_layout_passes=False),
    )


M_BLK = 400


def _mm_body(relu, s_ref, cnt_ref, x_ref, wl_ref, wr_ref, b_ref, o_ref):
    cnt = jnp.sum(cnt_ref[...], axis=1, keepdims=True)
    inv = 1.0 / jnp.maximum(cnt, 1.0)
    mean = s_ref[...] * inv
    acc = jnp.dot(mean, wl_ref[...], preferred_element_type=jnp.float32)
    acc = acc + jnp.dot(x_ref[...], wr_ref[...],
                        preferred_element_type=jnp.float32)
    acc = acc + b_ref[...]
    if relu:
        acc = jnp.maximum(acc, 0.0)
    o_ref[...] = acc


def _make_mm(d_in, relu):
    grid = N_NODES // M_BLK
    return pl.pallas_call(
        functools.partial(_mm_body, relu),
        grid=(grid,),
        in_specs=[
            pl.BlockSpec((M_BLK, d_in), lambda i: (i, 0)),
            pl.BlockSpec((M_BLK, TILES), lambda i: (i, 0)),
            pl.BlockSpec((M_BLK, d_in), lambda i: (i, 0)),
            pl.BlockSpec((d_in, D_HID), lambda i: (0, 0)),
            pl.BlockSpec((d_in, D_HID), lambda i: (0, 0)),
            pl.BlockSpec((1, D_HID), lambda i: (0, 0)),
        ],
        out_specs=pl.BlockSpec((M_BLK, D_HID), lambda i: (i, 0)),
        out_shape=jax.ShapeDtypeStruct((N_NODES, D_HID), jnp.float32),
    )


_mm1 = _make_mm(D_IN, True)
_mm2 = _make_mm(D_HID, False)


def kernel(x, edge_index, Wl1, bl1, Wr1, Wl2, bl2, Wr2):
    src = edge_index[0].astype(jnp.int32)
    dst = edge_index[1].astype(jnp.int32)
    pad = E_PAD - N_EDGES
    srcT = jnp.concatenate([src, jnp.zeros((pad,), jnp.int32)]
                           ).reshape(TILES, NB, BATCH)
    dstT = jnp.concatenate([dst, jnp.full((pad,), N_NODES, jnp.int32)]
                           ).reshape(TILES, NB, BATCH)

    xs = [x[:, i * CHUNK:(i + 1) * CHUNK] for i in range(D_IN // CHUNK)]
    *s1p, cntw = _make_seg_kernel(D_IN // (2 * CHUNK), True)(*xs, srcT, dstT)
    s1 = jnp.concatenate([p[:N_NODES] for p in s1p], axis=1)
    cnt = cntw[:, :N_NODES].T

    h = _mm1(s1, cnt, x, Wl1.T, Wr1.T, bl1.reshape(1, -1))

    hs = [h[:, i * CHUNK:(i + 1) * CHUNK] for i in range(D_HID // CHUNK)]
    s2p = _make_seg_kernel(D_HID // (2 * CHUNK), False)(*hs, srcT, dstT)
    s2 = jnp.concatenate([p[:N_NODES] for p in s2p], axis=1)

    return _mm2(s2, cnt, h, Wl2.T, Wr2.T, bl2.reshape(1, -1))


# speedup vs baseline: 3.2642x; 1.1775x over previous
import functools

import jax
import jax.numpy as jnp
from jax import lax
from jax.experimental import pallas as pl
from jax.experimental.pallas import tpu as pltpu
from jax.experimental.pallas import tpu_sc as plsc

N_NODES = 10000
N_EDGES = 160000
D_IN = 256
D_HID = 512

TILES = 16
BATCH = 128
NB = 80
W = 8
NW = NB // W
E_PAD = TILES * NB * BATCH
N_PAD = 10240
ROWS_PER_TILE = N_PAD // TILES
CHUNK = 128


def _agg_pass(tbl_hbm, out_hbm, srcT, dstT, src_w, dst_w, rows,
              sem_g, sem_s, sem_i, acc_sp, cnt_local, sub):
    base = sub * ROWS_PER_TILE

    def zrow(i, _):
        for k in range(CHUNK // 16):
            rows[0, i, pl.ds(k * 16, 16)] = jnp.zeros((16,), jnp.float32)
        return 0
    lax.fori_loop(0, BATCH, zrow, 0)
    for k in range(ROWS_PER_TILE // BATCH):
        pltpu.sync_copy(rows.at[0], acc_sp.at[pl.ds(base + k * BATCH, BATCH)])
    plsc.subcore_barrier()

    pltpu.async_copy(srcT.at[sub, pl.ds(0, W)], src_w.at[0], sem_i)
    pltpu.async_copy(dstT.at[sub, pl.ds(0, W)], dst_w.at[0], sem_i)

    def win(w, _):
        wb = lax.rem(w, 2)
        nwb = lax.rem(w + 1, 2)
        pltpu.make_async_copy(srcT.at[sub, pl.ds(w * W, W)], src_w.at[wb],
                              sem_i).wait()
        pltpu.make_async_copy(dstT.at[sub, pl.ds(w * W, W)], dst_w.at[wb],
                              sem_i).wait()

        @pl.when(w + 1 < NW)
        def _():
            pltpu.async_copy(srcT.at[sub, pl.ds((w + 1) * W, W)],
                             src_w.at[nwb], sem_i)
            pltpu.async_copy(dstT.at[sub, pl.ds((w + 1) * W, W)],
                             dst_w.at[nwb], sem_i)

        gd = [None] * W
        sd = [None] * W
        gd[0] = pltpu.async_copy(tbl_hbm.at[src_w.at[wb, 0]], rows.at[0],
                                 sem_g[0])
        for b in range(W):
            buf = b % 2
            if b + 1 < W:
                nbuf = (b + 1) % 2
                if b >= 1:
                    sd[b - 1].wait()
                gd[b + 1] = pltpu.async_copy(
                    tbl_hbm.at[src_w.at[wb, b + 1]], rows.at[nbuf],
                    sem_g[nbuf])
            if cnt_local is not None:
                ones16 = jnp.ones((16,), jnp.float32)
                for k in range(BATCH // 16):
                    d = dst_w[wb, b, pl.ds(k * 16, 16)]
                    plsc.addupdate_scatter(cnt_local, [d], ones16)
            gd[b].wait()
            sd[b] = pltpu.async_copy(rows.at[buf],
                                     acc_sp.at[dst_w.at[wb, b]],
                                     sem_s[buf], add=True)
        sd[W - 2].wait()
        sd[W - 1].wait()
        return 0
    lax.fori_loop(0, NW, win, 0)

    plsc.subcore_barrier()
    pltpu.sync_copy(acc_sp.at[pl.ds(base, ROWS_PER_TILE)],
                    out_hbm.at[pl.ds(base, ROWS_PER_TILE)])


def _seg_body(chunks_per_core, with_cnt, *refs):
    n_tbl = 2 * chunks_per_core
    tbls = refs[:n_tbl]
    srcT, dstT = refs[n_tbl], refs[n_tbl + 1]
    outs = refs[n_tbl + 2:2 * n_tbl + 2]
    rest = refs[2 * n_tbl + 2:]
    if with_cnt:
        cnt_hbm = rest[0]
        (src_w, dst_w, rows, sem_g0, sem_g1, sem_s0, sem_s1, sem_i,
         cnt_local, acc_sp) = rest[1:]
    else:
        cnt_hbm = None
        (src_w, dst_w, rows, sem_g0, sem_g1, sem_s0, sem_s1, sem_i,
         acc_sp) = rest
        cnt_local = None
    sem_g = (sem_g0, sem_g1)
    sem_s = (sem_s0, sem_s1)

    core = lax.axis_index("c")
    sub = lax.axis_index("s")

    if with_cnt:
        def zc(i, _):
            cnt_local[pl.ds(i * 16, 16)] = jnp.zeros((16,), jnp.float32)
            return 0
        lax.fori_loop(0, N_PAD // 16, zc, 0)

    @pl.when(core == 0)
    def _():
        for ci in range(chunks_per_core):
            _agg_pass(tbls[ci], outs[ci], srcT, dstT, src_w, dst_w, rows,
                      sem_g, sem_s, sem_i, acc_sp,
                      cnt_local if (with_cnt and ci == 0) else None, sub)
        if with_cnt:
            pltpu.sync_copy(cnt_local, cnt_hbm.at[sub])

    @pl.when(core == 1)
    def _():
        for ci in range(chunks_per_core):
            _agg_pass(tbls[chunks_per_core + ci], outs[chunks_per_core + ci],
                      srcT, dstT, src_w, dst_w, rows, sem_g, sem_s, sem_i,
                      acc_sp, None, sub)


@functools.lru_cache(maxsize=None)
def _make_seg_kernel(chunks_per_core, with_cnt):
    mesh = plsc.VectorSubcoreMesh(core_axis_name="c", subcore_axis_name="s")
    n_tbl = 2 * chunks_per_core
    out_type = [jax.ShapeDtypeStruct((N_PAD, CHUNK), jnp.float32)
                for _ in range(n_tbl)]
    if with_cnt:
        out_type.append(jax.ShapeDtypeStruct((TILES, N_PAD), jnp.float32))
    scratch = [
        pltpu.VMEM((2, W, BATCH), jnp.int32),
        pltpu.VMEM((2, W, BATCH), jnp.int32),
        pltpu.VMEM((2, BATCH, CHUNK), jnp.float32),
        pltpu.SemaphoreType.DMA,
        pltpu.SemaphoreType.DMA,
        pltpu.SemaphoreType.DMA,
        pltpu.SemaphoreType.DMA,
        pltpu.SemaphoreType.DMA,
    ]
    if with_cnt:
        scratch.append(pltpu.VMEM((N_PAD,), jnp.float32))
    scratch.append(pltpu.VMEM_SHARED((N_PAD, CHUNK), jnp.float32))
    return pl.kernel(
        functools.partial(_seg_body, chunks_per_core, with_cnt),
        out_type=tuple(out_type),
        mesh=mesh,
        scratch_types=tuple(scratch),
        compiler_params=pltpu.CompilerParams(needs_layout_passes=False),
    )


M_BLK = 400


def _mm_body(relu, s_ref, cnt_ref, x_ref, wl_ref, wr_ref, b_ref, o_ref):
    cnt = jnp.sum(cnt_ref[...], axis=1, keepdims=True)
    inv = 1.0 / jnp.maximum(cnt, 1.0)
    mean = s_ref[...] * inv
    acc = jnp.dot(mean, wl_ref[...], preferred_element_type=jnp.float32)
    acc = acc + jnp.dot(x_ref[...], wr_ref[...],
                        preferred_element_type=jnp.float32)
    acc = acc + b_ref[...]
    if relu:
        acc = jnp.maximum(acc, 0.0)
    o_ref[...] = acc


def _make_mm(d_in, relu):
    grid = N_NODES // M_BLK
    return pl.pallas_call(
        functools.partial(_mm_body, relu),
        grid=(grid,),
        in_specs=[
            pl.BlockSpec((M_BLK, d_in), lambda i: (i, 0)),
            pl.BlockSpec((M_BLK, TILES), lambda i: (i, 0)),
            pl.BlockSpec((M_BLK, d_in), lambda i: (i, 0)),
            pl.BlockSpec((d_in, D_HID), lambda i: (0, 0)),
            pl.BlockSpec((d_in, D_HID), lambda i: (0, 0)),
            pl.BlockSpec((1, D_HID), lambda i: (0, 0)),
        ],
        out_specs=pl.BlockSpec((M_BLK, D_HID), lambda i: (i, 0)),
        out_shape=jax.ShapeDtypeStruct((N_NODES, D_HID), jnp.float32),
    )


_mm1 = _make_mm(D_IN, True)
_mm2 = _make_mm(D_HID, False)


def kernel(x, edge_index, Wl1, bl1, Wr1, Wl2, bl2, Wr2):
    src = edge_index[0].astype(jnp.int32)
    dst = edge_index[1].astype(jnp.int32)
    pad = E_PAD - N_EDGES
    srcT = jnp.concatenate([src, jnp.zeros((pad,), jnp.int32)]
                           ).reshape(TILES, NB, BATCH)
    dstT = jnp.concatenate([dst, jnp.full((pad,), N_NODES, jnp.int32)]
                           ).reshape(TILES, NB, BATCH)

    xs = [x[:, i * CHUNK:(i + 1) * CHUNK] for i in range(D_IN // CHUNK)]
    *s1p, cntw = _make_seg_kernel(D_IN // (2 * CHUNK), True)(*xs, srcT, dstT)
    s1 = jnp.concatenate([p[:N_NODES] for p in s1p], axis=1)
    cnt = cntw[:, :N_NODES].T

    h = _mm1(s1, cnt, x, Wl1.T, Wr1.T, bl1.reshape(1, -1))

    hs = [h[:, i * CHUNK:(i + 1) * CHUNK] for i in range(D_HID // CHUNK)]
    s2p = _make_seg_kernel(D_HID // (2 * CHUNK), False)(*hs, srcT, dstT)
    s2 = jnp.concatenate([p[:N_NODES] for p in s2p], axis=1)

    return _mm2(s2, cnt, h, Wl2.T, Wr2.T, bl2.reshape(1, -1))


# speedup vs baseline: 3.3721x; 1.0331x over previous
import functools

import jax
import jax.numpy as jnp
from jax import lax
from jax.experimental import pallas as pl
from jax.experimental.pallas import tpu as pltpu
from jax.experimental.pallas import tpu_sc as plsc

N_NODES = 10000
N_EDGES = 160000
D_IN = 256
D_HID = 512

TILES = 16
BATCH = 128
NB = 80
E_PAD = TILES * NB * BATCH
N_PAD = 10240
ROWS_PER_TILE = N_PAD // TILES
CHUNK = 128
ZR = 32


def _agg_pass(tbl_hbm, out_hbm, srcT, dstT, src_w, dst_w, rows, zbuf,
              sem_g, sem_s, sem_i, acc_sp, cnt_local, cnt_lo, cnt_hi,
              w_sz, sub):
    base = sub * ROWS_PER_TILE
    nw = NB // w_sz

    if zbuf is None:
        def zrow(i, _):
            for k in range(CHUNK // 16):
                rows[0, i, pl.ds(k * 16, 16)] = jnp.zeros((16,), jnp.float32)
            return 0
        lax.fori_loop(0, BATCH, zrow, 0)
        for k in range(ROWS_PER_TILE // BATCH):
            pltpu.sync_copy(rows.at[0],
                            acc_sp.at[pl.ds(base + k * BATCH, BATCH)])
    else:
        for k in range(ROWS_PER_TILE // ZR):
            pltpu.sync_copy(zbuf, acc_sp.at[pl.ds(base + k * ZR, ZR)])
    plsc.subcore_barrier()

    pltpu.async_copy(srcT.at[sub, pl.ds(0, w_sz)], src_w.at[0], sem_i)
    pltpu.async_copy(dstT.at[sub, pl.ds(0, w_sz)], dst_w.at[0], sem_i)

    def win(w, _):
        wb = lax.rem(w, 2)
        nwb = lax.rem(w + 1, 2)
        pltpu.make_async_copy(srcT.at[sub, pl.ds(w * w_sz, w_sz)],
                              src_w.at[wb], sem_i).wait()
        pltpu.make_async_copy(dstT.at[sub, pl.ds(w * w_sz, w_sz)],
                              dst_w.at[wb], sem_i).wait()

        @pl.when(w + 1 < nw)
        def _():
            pltpu.async_copy(srcT.at[sub, pl.ds((w + 1) * w_sz, w_sz)],
                             src_w.at[nwb], sem_i)
            pltpu.async_copy(dstT.at[sub, pl.ds((w + 1) * w_sz, w_sz)],
                             dst_w.at[nwb], sem_i)

        gd = [None] * w_sz
        sd = [None] * w_sz
        gd[0] = pltpu.async_copy(tbl_hbm.at[src_w.at[wb, 0]], rows.at[0],
                                 sem_g[0])
        for b in range(w_sz):
            buf = b % 2
            if b + 1 < w_sz:
                nbuf = (b + 1) % 2
                if b >= 1:
                    sd[b - 1].wait()
                gd[b + 1] = pltpu.async_copy(
                    tbl_hbm.at[src_w.at[wb, b + 1]], rows.at[nbuf],
                    sem_g[nbuf])
            if cnt_local is not None:
                @pl.when((w >= cnt_lo) & (w < cnt_hi))
                def _():
                    ones16 = jnp.ones((16,), jnp.float32)
                    for k in range(BATCH // 16):
                        d = dst_w[wb, b, pl.ds(k * 16, 16)]
                        plsc.addupdate_scatter(cnt_local, [d], ones16)
            gd[b].wait()
            sd[b] = pltpu.async_copy(rows.at[buf],
                                     acc_sp.at[dst_w.at[wb, b]],
                                     sem_s[buf], add=True)
        sd[w_sz - 2].wait()
        sd[w_sz - 1].wait()
        return 0
    lax.fori_loop(0, nw, win, 0)

    plsc.subcore_barrier()
    pltpu.sync_copy(acc_sp.at[pl.ds(base, ROWS_PER_TILE)],
                    out_hbm.at[pl.ds(base, ROWS_PER_TILE)])


def _seg_body(chunks_per_core, with_cnt, w_sz, with_zbuf, *refs):
    n_tbl = 2 * chunks_per_core
    tbls = refs[:n_tbl]
    srcT, dstT = refs[n_tbl], refs[n_tbl + 1]
    outs = refs[n_tbl + 2:2 * n_tbl + 2]
    rest = list(refs[2 * n_tbl + 2:])
    cnt_hbm = rest.pop(0) if with_cnt else None
    src_w, dst_w, rows = rest[0], rest[1], rest[2]
    sem_g = (rest[3], rest[4])
    sem_s = (rest[5], rest[6])
    sem_i = rest[7]
    rest = rest[8:]
    cnt_local = rest.pop(0) if with_cnt else None
    zbuf = rest.pop(0) if with_zbuf else None
    acc_sp = rest.pop(0)

    core = lax.axis_index("c")
    sub = lax.axis_index("s")
    nw = NB // w_sz

    if with_cnt:
        def zc(i, _):
            cnt_local[pl.ds(i * 16, 16)] = jnp.zeros((16,), jnp.float32)
            return 0
        lax.fori_loop(0, N_PAD // 16, zc, 0)
    if with_zbuf:
        def zb(i, _):
            for k in range(CHUNK // 16):
                zbuf[i, pl.ds(k * 16, 16)] = jnp.zeros((16,), jnp.float32)
            return 0
        lax.fori_loop(0, ZR, zb, 0)

    @pl.when(core == 0)
    def _():
        for ci in range(chunks_per_core):
            _agg_pass(tbls[ci], outs[ci], srcT, dstT, src_w, dst_w, rows,
                      zbuf, sem_g, sem_s, sem_i, acc_sp,
                      cnt_local if (with_cnt and ci == 0) else None,
                      0, nw // 2, w_sz, sub)
        if with_cnt:
            pltpu.sync_copy(cnt_local, cnt_hbm.at[0, sub])

    @pl.when(core == 1)
    def _():
        for ci in range(chunks_per_core):
            _agg_pass(tbls[chunks_per_core + ci], outs[chunks_per_core + ci],
                      srcT, dstT, src_w, dst_w, rows, zbuf, sem_g, sem_s,
                      sem_i, acc_sp,
                      cnt_local if (with_cnt and ci == 0) else None,
                      nw // 2, nw, w_sz, sub)
        if with_cnt:
            pltpu.sync_copy(cnt_local, cnt_hbm.at[1, sub])


@functools.lru_cache(maxsize=None)
def _make_seg_kernel(chunks_per_core, with_cnt, w_sz, with_zbuf):
    mesh = plsc.VectorSubcoreMesh(core_axis_name="c", subcore_axis_name="s")
    n_tbl = 2 * chunks_per_core
    out_type = [jax.ShapeDtypeStruct((N_PAD, CHUNK), jnp.float32)
                for _ in range(n_tbl)]
    if with_cnt:
        out_type.append(jax.ShapeDtypeStruct((2, TILES, N_PAD), jnp.float32))
    scratch = [
        pltpu.VMEM((2, w_sz, BATCH), jnp.int32),
        pltpu.VMEM((2, w_sz, BATCH), jnp.int32),
        pltpu.VMEM((2, BATCH, CHUNK), jnp.float32),
        pltpu.SemaphoreType.DMA,
        pltpu.SemaphoreType.DMA,
        pltpu.SemaphoreType.DMA,
        pltpu.SemaphoreType.DMA,
        pltpu.SemaphoreType.DMA,
    ]
    if with_cnt:
        scratch.append(pltpu.VMEM((N_PAD,), jnp.float32))
    if with_zbuf:
        scratch.append(pltpu.VMEM((ZR, CHUNK), jnp.float32))
    scratch.append(pltpu.VMEM_SHARED((N_PAD, CHUNK), jnp.float32))
    return pl.kernel(
        functools.partial(_seg_body, chunks_per_core, with_cnt, w_sz,
                          with_zbuf),
        out_type=tuple(out_type),
        mesh=mesh,
        scratch_types=tuple(scratch),
        compiler_params=pltpu.CompilerParams(needs_layout_passes=False),
    )


M_BLK = 400


def _mm1_body(sa_ref, sb_ref, cnt_ref, x_ref, wl_ref, wr_ref, b_ref, *outs):
    s1 = jnp.concatenate([sa_ref[...], sb_ref[...]], axis=1)
    cnt = jnp.sum(cnt_ref[...], axis=1, keepdims=True)
    inv = 1.0 / jnp.maximum(cnt, 1.0)
    mean = s1 * inv
    acc = jnp.dot(mean, wl_ref[...], preferred_element_type=jnp.float32)
    acc = acc + jnp.dot(x_ref[...], wr_ref[...],
                        preferred_element_type=jnp.float32)
    h = jnp.maximum(acc + b_ref[...], 0.0)
    for c in range(4):
        outs[c][...] = h[:, c * CHUNK:(c + 1) * CHUNK]


def _make_mm1():
    grid = N_NODES // M_BLK
    return pl.pallas_call(
        _mm1_body,
        grid=(grid,),
        in_specs=[
            pl.BlockSpec((M_BLK, CHUNK), lambda i: (i, 0)),
            pl.BlockSpec((M_BLK, CHUNK), lambda i: (i, 0)),
            pl.BlockSpec((M_BLK, 2 * TILES), lambda i: (i, 0)),
            pl.BlockSpec((M_BLK, D_IN), lambda i: (i, 0)),
            pl.BlockSpec((D_IN, D_HID), lambda i: (0, 0)),
            pl.BlockSpec((D_IN, D_HID), lambda i: (0, 0)),
            pl.BlockSpec((1, D_HID), lambda i: (0, 0)),
        ],
        out_specs=[pl.BlockSpec((M_BLK, CHUNK), lambda i: (i, 0))
                   for _ in range(4)],
        out_shape=[jax.ShapeDtypeStruct((N_NODES, CHUNK), jnp.float32)
                   for _ in range(4)],
    )


def _mm2_body(t0, t1, t2, t3, cnt_ref, h0, h1, h2, h3,
              wl_ref, wr_ref, b_ref, o_ref):
    s2 = jnp.concatenate([t0[...], t1[...], t2[...], t3[...]], axis=1)
    h = jnp.concatenate([h0[...], h1[...], h2[...], h3[...]], axis=1)
    cnt = jnp.sum(cnt_ref[...], axis=1, keepdims=True)
    inv = 1.0 / jnp.maximum(cnt, 1.0)
    mean = s2 * inv
    acc = jnp.dot(mean, wl_ref[...], preferred_element_type=jnp.float32)
    acc = acc + jnp.dot(h, wr_ref[...], preferred_element_type=jnp.float32)
    o_ref[...] = acc + b_ref[...]


def _make_mm2():
    grid = N_NODES // M_BLK
    return pl.pallas_call(
        _mm2_body,
        grid=(grid,),
        in_specs=(
            [pl.BlockSpec((M_BLK, CHUNK), lambda i: (i, 0))
             for _ in range(4)]
            + [pl.BlockSpec((M_BLK, 2 * TILES), lambda i: (i, 0))]
            + [pl.BlockSpec((M_BLK, CHUNK), lambda i: (i, 0))
               for _ in range(4)]
            + [
                pl.BlockSpec((D_HID, D_HID), lambda i: (0, 0)),
                pl.BlockSpec((D_HID, D_HID), lambda i: (0, 0)),
                pl.BlockSpec((1, D_HID), lambda i: (0, 0)),
            ]
        ),
        out_specs=pl.BlockSpec((M_BLK, D_HID), lambda i: (i, 0)),
        out_shape=jax.ShapeDtypeStruct((N_NODES, D_HID), jnp.float32),
    )


_mm1 = _make_mm1()
_mm2 = _make_mm2()


def kernel(x, edge_index, Wl1, bl1, Wr1, Wl2, bl2, Wr2):
    src = edge_index[0].astype(jnp.int32)
    dst = edge_index[1].astype(jnp.int32)
    pad = E_PAD - N_EDGES
    srcT = jnp.concatenate([src, jnp.zeros((pad,), jnp.int32)]
                           ).reshape(TILES, NB, BATCH)
    dstT = jnp.concatenate([dst, jnp.full((pad,), N_NODES, jnp.int32)]
                           ).reshape(TILES, NB, BATCH)

    xa = x[:, :CHUNK]
    xb = x[:, CHUNK:]
    sa, sb, cntw = _make_seg_kernel(1, True, 8, False)(xa, xb, srcT, dstT)
    cntT = cntw.reshape(2 * TILES, N_PAD)[:, :N_NODES].T

    h0, h1, h2, h3 = _mm1(sa, sb, cntT, x, Wl1.T, Wr1.T, bl1.reshape(1, -1))

    t0, t1, t2, t3 = _make_seg_kernel(2, False, 16, True)(h0, h1, h2, h3,
                                                          srcT, dstT)

    return _mm2(t0, t1, t2, t3, cntT, h0, h1, h2, h3,
                Wl2.T, Wr2.T, bl2.reshape(1, -1))
